# baseline (speedup 1.0000x reference)
import functools

import jax
import jax.numpy as jnp
from jax.experimental import pallas as pl
from jax.experimental.pallas import tpu as pltpu

B = 1024
F = 128
BLOCK_N = 2048


def _matmul_kernel(x_ref, m_ref, o_ref):
    o_ref[...] = jax.lax.dot_general(
        x_ref[...],
        m_ref[...],
        dimension_numbers=(((1,), (1,)), ((), ())),
        preferred_element_type=jnp.float32,
    )


@jax.jit
def _logits(inputs, mem):
    n = mem.shape[0]
    grid = (pl.cdiv(n, BLOCK_N),)
    return pl.pallas_call(
        _matmul_kernel,
        grid=grid,
        in_specs=[
            pl.BlockSpec((B, F), lambda i: (0, 0)),
            pl.BlockSpec((BLOCK_N, F), lambda i: (i, 0)),
        ],
        out_specs=pl.BlockSpec((B, BLOCK_N), lambda i: (0, i)),
        out_shape=jax.ShapeDtypeStruct((B, n), jnp.float32),
        compiler_params=pltpu.CompilerParams(
            dimension_semantics=("parallel",),
        ),
    )(inputs, mem)


def kernel(inputs, targets, epoch, mem):
    del targets, epoch
    return _logits(inputs, mem)


# speedup vs baseline: 1.0012x; 1.0012x over previous
import functools

import jax
import jax.numpy as jnp
from jax.experimental import pallas as pl
from jax.experimental.pallas import tpu as pltpu

B = 1024
F = 128
BLOCK_N = 2176


def _matmul_kernel(x_ref, m_ref, o_ref):
    o_ref[...] = jax.lax.dot_general(
        x_ref[...],
        m_ref[...],
        dimension_numbers=(((1,), (1,)), ((), ())),
        preferred_element_type=jnp.float32,
    )


@jax.jit
def _logits(inputs, mem):
    n = mem.shape[0]
    grid = (pl.cdiv(n, BLOCK_N),)
    return pl.pallas_call(
        _matmul_kernel,
        grid=grid,
        in_specs=[
            pl.BlockSpec((B, F), lambda i: (0, 0)),
            pl.BlockSpec((BLOCK_N, F), lambda i: (i, 0)),
        ],
        out_specs=pl.BlockSpec((B, BLOCK_N), lambda i: (0, i)),
        out_shape=jax.ShapeDtypeStruct((B, n), jnp.float32),
        compiler_params=pltpu.CompilerParams(
            dimension_semantics=("parallel",),
        ),
    )(inputs, mem)


def kernel(inputs, targets, epoch, mem):
    del targets, epoch
    return _logits(inputs, mem)


# speedup vs baseline: 1.1205x; 1.1191x over previous
import jax
import jax.numpy as jnp
from jax.experimental import pallas as pl
from jax.experimental.pallas import tpu as pltpu

B = 1024
F = 128
N = 100000
BLOCK_N = 2176
NBLK = (N + BLOCK_N - 1) // BLOCK_N
LAST_I = NBLK - 1
LAST_W = 2048
TAIL_ROW0 = N - F - LAST_I * BLOCK_N


def _dot(x, m):
    return jax.lax.dot_general(
        x,
        m,
        dimension_numbers=(((1,), (1,)), ((), ())),
        preferred_element_type=jnp.float32,
    )


def _mm_kernel(x_ref, m_ref, o_ref, t_ref, acc_ref, sem_ref):
    i = pl.program_id(0)
    slot = jax.lax.rem(i, 2)

    @pl.when(i >= 2)
    def _wait_prev():
        pltpu.make_async_copy(
            acc_ref.at[slot],
            o_ref.at[:, pl.ds((i - 2) * BLOCK_N, BLOCK_N)],
            sem_ref.at[slot],
        ).wait()

    acc_ref[slot] = _dot(x_ref[...], m_ref[...])

    @pl.when(i < LAST_I)
    def _copy_full():
        pltpu.make_async_copy(
            acc_ref.at[slot],
            o_ref.at[:, pl.ds(i * BLOCK_N, BLOCK_N)],
            sem_ref.at[slot],
        ).start()

    @pl.when(i == LAST_I)
    def _finish():
        main = pltpu.make_async_copy(
            acc_ref.at[slot, :, pl.ds(0, LAST_W)],
            o_ref.at[:, pl.ds(i * BLOCK_N, LAST_W)],
            sem_ref.at[slot],
        )
        main.start()
        pltpu.make_async_copy(
            acc_ref.at[1 - slot],
            o_ref.at[:, pl.ds((i - 1) * BLOCK_N, BLOCK_N)],
            sem_ref.at[1 - slot],
        ).wait()
        acc_ref[1 - slot, :, : F] = _dot(x_ref[...], m_ref[pl.ds(TAIL_ROW0, F), :])
        tail = pltpu.make_async_copy(
            acc_ref.at[1 - slot, :, pl.ds(0, F)],
            t_ref.at[...],
            sem_ref.at[1 - slot],
        )
        tail.start()
        main.wait()
        tail.wait()


@jax.jit
def _logits(inputs, mem):
    out, tail = pl.pallas_call(
        _mm_kernel,
        grid=(NBLK,),
        in_specs=[
            pl.BlockSpec((B, F), lambda i: (0, 0)),
            pl.BlockSpec((BLOCK_N, F), lambda i: (i, 0)),
        ],
        out_specs=[
            pl.BlockSpec(memory_space=pltpu.MemorySpace.HBM),
            pl.BlockSpec(memory_space=pltpu.MemorySpace.HBM),
        ],
        out_shape=[
            jax.ShapeDtypeStruct((B, N), jnp.float32),
            jax.ShapeDtypeStruct((B, F), jnp.float32),
        ],
        scratch_shapes=[
            pltpu.VMEM((2, B, BLOCK_N), jnp.float32),
            pltpu.SemaphoreType.DMA((2,)),
        ],
        compiler_params=pltpu.CompilerParams(
            dimension_semantics=("arbitrary",),
        ),
    )(inputs, mem)
    return jax.lax.dynamic_update_slice(out, tail, (0, N - F))


def kernel(inputs, targets, epoch, mem):
    del targets, epoch
    return _logits(inputs, mem)
